# baseline (speedup 1.0000x reference)
import functools

import jax
import jax.numpy as jnp
from jax import lax
from jax.experimental import pallas as pl
from jax.experimental.pallas import tpu as pltpu
from jax.experimental.pallas import tpu_sc as plsc

_B = 16384
_D = 128
_NW = 32
_R = _B // _NW
_C = 128
_NCHUNK = _R // _C
_L = 16
_G = _C // _L
_EPS = 1e-6
_MARGIN = 1.0

_sc_mesh = plsc.VectorSubcoreMesh(core_axis_name="c", subcore_axis_name="s")


@functools.partial(
    pl.kernel,
    out_type=(
        jax.ShapeDtypeStruct((_B,), jnp.float32),
        jax.ShapeDtypeStruct((_B,), jnp.float32),
    ),
    mesh=_sc_mesh,
    compiler_params=pltpu.CompilerParams(needs_layout_passes=False),
    scratch_types=[
        pltpu.VMEM((3 * _R,), jnp.int32),
        pltpu.VMEM((2, 3 * _C, _D), jnp.float32),
        pltpu.VMEM((_C,), jnp.float32),
        pltpu.VMEM((_C,), jnp.float32),
        pltpu.SemaphoreType.DMA,
        pltpu.SemaphoreType.DMA,
    ],
)
def _sc_distances(a_hbm, p_hbm, n_hbm, table_hbm, dp_hbm, dn_hbm,
                  idx_all, rows_b, dp_v, dn_v, sem0, sem1):
    wid = lax.axis_index("s") * 2 + lax.axis_index("c")
    base = wid * _R
    for c in range(_NCHUNK):
        cb = c * 3 * _C
        pltpu.sync_copy(a_hbm.at[pl.ds(base + c * _C, _C)],
                        idx_all.at[pl.ds(cb, _C)])
        pltpu.sync_copy(p_hbm.at[pl.ds(base + c * _C, _C)],
                        idx_all.at[pl.ds(cb + _C, _C)])
        pltpu.sync_copy(n_hbm.at[pl.ds(base + c * _C, _C)],
                        idx_all.at[pl.ds(cb + 2 * _C, _C)])

    lanes = lax.iota(jnp.int32, _L)
    sems = (sem0, sem1)

    def start_chunk(c):
        b = c % 2
        sl = pl.ds(c * 3 * _C, 3 * _C)
        return pltpu.async_copy(table_hbm.at[idx_all.at[sl]], rows_b.at[b],
                                sems[b])

    handle = start_chunk(0)
    for c in range(_NCHUNK):
        b = c % 2
        if c + 1 < _NCHUNK:
            next_handle = start_chunk(c + 1)
        handle.wait()
        if c + 1 < _NCHUNK:
            handle = next_handle
        rows_c = rows_b.at[b]

        def group_body(g, carry):
            res_p = jnp.zeros((_L,), jnp.float32)
            res_n = jnp.zeros((_L,), jnp.float32)
            for j in range(_L):
                r = g * _L + j
                acc_p = jnp.zeros((_L,), jnp.float32)
                acc_n = jnp.zeros((_L,), jnp.float32)
                for s in range(_D // _L):
                    sl2 = pl.ds(s * _L, _L)
                    va = rows_c[r, sl2]
                    vp = rows_c[_C + r, sl2]
                    vn = rows_c[2 * _C + r, sl2]
                    tp = va - vp + _EPS
                    tn = va - vn + _EPS
                    acc_p = acc_p + tp * tp
                    acc_n = acc_n + tn * tn
                res_p = jnp.where(lanes == j, jnp.sum(acc_p), res_p)
                res_n = jnp.where(lanes == j, jnp.sum(acc_n), res_n)
            rows = g * _L + lanes
            plsc.store_scatter(dp_v, [rows], res_p)
            plsc.store_scatter(dn_v, [rows], res_n)
            return carry

        lax.fori_loop(0, _G, group_body, 0)

        pltpu.sync_copy(dp_v, dp_hbm.at[pl.ds(base + c * _C, _C)])
        pltpu.sync_copy(dn_v, dn_hbm.at[pl.ds(base + c * _C, _C)])


def _tc_loss(dp_ref, dn_ref, out_ref):
    d_pos = jnp.sqrt(dp_ref[...])
    d_neg = jnp.sqrt(dn_ref[...])
    hinge = jnp.maximum(d_pos - d_neg + _MARGIN, 0.0)
    out_ref[0, 0] = jnp.sum(hinge) * (1.0 / _B)


_tc_call = pl.pallas_call(
    _tc_loss,
    out_shape=jax.ShapeDtypeStruct((1, 1), jnp.float32),
    out_specs=pl.BlockSpec(memory_space=pltpu.SMEM),
)


def kernel(a, p, n, table):
    a = a.astype(jnp.int32)
    p = p.astype(jnp.int32)
    n = n.astype(jnp.int32)
    dp_sq, dn_sq = _sc_distances(a, p, n, table)
    out = _tc_call(dp_sq.reshape(_B // _D, _D), dn_sq.reshape(_B // _D, _D))
    return out[0, 0]
